# baseline (speedup 1.0000x reference)
import functools

import numpy as _np

import jax
import jax.numpy as jnp
from jax import lax
from jax.experimental import pallas as pl
from jax.experimental.pallas import tpu as pltpu
from jax.experimental.pallas import tpu_sc as plsc

B = 4096
CTX = 20
D = 64
V = 100000

NC = 2
NS = 16
NW = NC * NS
RPW = B // NW
RPC = 4
IPC = RPC * CTX
NCHUNK = RPW // RPC
LG = D // 16


def _sc_pool_body(ctx_hbm, table_hbm, out_hbm, ctx_full, idx_v, rows0, rows1,
                  pooled_v, sem0, sem1):
    wid = lax.axis_index("s") * NC + lax.axis_index("c")
    pltpu.sync_copy(ctx_hbm, ctx_full)
    rbase = wid * RPW
    lanes = lax.iota(jnp.int32, 16)
    rq, jq = [], []
    for m in range(5):
        lo = (16 * m) // CTX
        th = (lo + 1) * CTX - 16 * m
        r_m = jnp.where(lanes >= th, lo + 1, lo)
        rq.append(r_m)
        jq.append(16 * m + lanes - r_m * CTX)

    def tr(c, carry):
        for m in range(5):
            idx_v[c, pl.ds(m * 16, 16)] = plsc.load_gather(
                ctx_full, [jq[m], rbase + RPC * c + rq[m]])
        return carry

    lax.fori_loop(0, NCHUNK, tr, 0)

    def accum(rows_v, c):
        for r in range(RPC):
            for g in range(LG):
                sl = pl.ds(g * 16, 16)
                acc = rows_v[r * CTX, sl]
                for j in range(1, CTX):
                    acc = acc + rows_v[r * CTX + j, sl]
                pooled_v[c * RPC + r, sl] = acc * (1.0 / CTX)

    def start(c, rows_v, sem):
        return pltpu.async_copy(table_hbm.at[idx_v.at[c]], rows_v, sem)

    start(0, rows0, sem0)
    start(1, rows1, sem1)

    def pair(i, carry):
        pltpu.make_async_copy(
            table_hbm.at[idx_v.at[0]], rows0, sem0).wait()
        accum(rows0, 2 * i)
        start(2 * i + 2, rows0, sem0)
        pltpu.make_async_copy(
            table_hbm.at[idx_v.at[0]], rows1, sem1).wait()
        accum(rows1, 2 * i + 1)
        start(2 * i + 3, rows1, sem1)
        return carry

    lax.fori_loop(0, NCHUNK // 2 - 1, pair, 0)
    pltpu.make_async_copy(table_hbm.at[idx_v.at[0]], rows0, sem0).wait()
    accum(rows0, NCHUNK - 2)
    pltpu.make_async_copy(table_hbm.at[idx_v.at[0]], rows1, sem1).wait()
    accum(rows1, NCHUNK - 1)
    pltpu.sync_copy(pooled_v, out_hbm.at[pl.ds(wid * RPW, RPW)])


_sc_pool = functools.partial(
    pl.kernel,
    out_type=jax.ShapeDtypeStruct((B, D), jnp.float32),
    mesh=plsc.VectorSubcoreMesh(core_axis_name="c", subcore_axis_name="s"),
    compiler_params=pltpu.CompilerParams(use_tc_tiling_on_sc=False, needs_layout_passes=False),
    scratch_types=[
        pltpu.VMEM((CTX, B), jnp.int32),
        pltpu.VMEM((NCHUNK, IPC), jnp.int32),
        pltpu.VMEM((IPC, D), jnp.float32),
        pltpu.VMEM((IPC, D), jnp.float32),
        pltpu.VMEM((RPW, D), jnp.float32),
        pltpu.SemaphoreType.DMA,
        pltpu.SemaphoreType.DMA,
    ],
)(_sc_pool_body)


BN = 512
NB = (V + BN - 1) // BN


def _mm_body(wt_ref, p_ref, b_ref, o_ref):
    o_ref[...] = lax.dot_general(
        wt_ref[...], p_ref[...],
        (((0,), (1,)), ((), ())),
        preferred_element_type=jnp.float32,
    ) + jnp.transpose(b_ref[...])


def _tc_linear(W_t, pooled, b2):
    return pl.pallas_call(
        _mm_body,
        grid=(NB,),
        in_specs=[
            pl.BlockSpec((D, BN), lambda i: (0, i)),
            pl.BlockSpec((B, D), lambda i: (0, 0)),
            pl.BlockSpec((1, BN), lambda i: (0, i)),
        ],
        out_specs=pl.BlockSpec((BN, B), lambda i: (i, 0)),
        out_shape=jax.ShapeDtypeStruct((V, B), jnp.float32),
    )(W_t, pooled, b2)


def kernel(context, emb_table, W, b):
    ctx_t = context.astype(jnp.int32).T
    pooled = _sc_pool(ctx_t, emb_table)
    out_t = _tc_linear(W.T, pooled, b.reshape(1, V))
    return out_t.T


# speedup vs baseline: 1.0117x; 1.0117x over previous
import functools

import numpy as _np

import jax
import jax.numpy as jnp
from jax import lax
from jax.experimental import pallas as pl
from jax.experimental.pallas import tpu as pltpu
from jax.experimental.pallas import tpu_sc as plsc

B = 4096
CTX = 20
D = 64
V = 100000

NC = 2
NS = 16
NW = NC * NS
RPW = B // NW
RPC = 4
IPC = RPC * CTX
NCHUNK = RPW // RPC
LG = D // 16


def _sc_pool_body(ctx_hbm, table_hbm, out_hbm, ctx_full, idx_v, rows0, rows1,
                  pooled_v, sem0, sem1):
    wid = lax.axis_index("s") * NC + lax.axis_index("c")
    pltpu.sync_copy(ctx_hbm, ctx_full)
    rbase = wid * RPW
    lanes = lax.iota(jnp.int32, 16)
    rq, jq = [], []
    for m in range(5):
        lo = (16 * m) // CTX
        th = (lo + 1) * CTX - 16 * m
        r_m = jnp.where(lanes >= th, lo + 1, lo)
        rq.append(r_m)
        jq.append(16 * m + lanes - r_m * CTX)

    def tr(c, carry):
        for m in range(5):
            idx_v[c, pl.ds(m * 16, 16)] = plsc.load_gather(
                ctx_full, [jq[m], rbase + RPC * c + rq[m]])
        return carry

    lax.fori_loop(0, NCHUNK, tr, 0)

    def accum(rows_v, c):
        for r in range(RPC):
            for g in range(LG):
                sl = pl.ds(g * 16, 16)
                acc = rows_v[r * CTX, sl]
                for j in range(1, CTX):
                    acc = acc + rows_v[r * CTX + j, sl]
                pooled_v[c * RPC + r, sl] = acc * (1.0 / CTX)

    def start(c, rows_v, sem):
        return pltpu.async_copy(table_hbm.at[idx_v.at[c]], rows_v, sem)

    start(0, rows0, sem0)
    start(1, rows1, sem1)

    def pair(i, carry):
        pltpu.make_async_copy(
            table_hbm.at[idx_v.at[0]], rows0, sem0).wait()
        accum(rows0, 2 * i)
        start(2 * i + 2, rows0, sem0)
        pltpu.make_async_copy(
            table_hbm.at[idx_v.at[0]], rows1, sem1).wait()
        accum(rows1, 2 * i + 1)
        start(2 * i + 3, rows1, sem1)
        return carry

    lax.fori_loop(0, NCHUNK // 2 - 1, pair, 0)
    pltpu.make_async_copy(table_hbm.at[idx_v.at[0]], rows0, sem0).wait()
    accum(rows0, NCHUNK - 2)
    pltpu.make_async_copy(table_hbm.at[idx_v.at[0]], rows1, sem1).wait()
    accum(rows1, NCHUNK - 1)
    pltpu.sync_copy(pooled_v, out_hbm.at[pl.ds(wid * RPW, RPW)])


_sc_pool = functools.partial(
    pl.kernel,
    out_type=jax.ShapeDtypeStruct((B, D), jnp.float32),
    mesh=plsc.VectorSubcoreMesh(core_axis_name="c", subcore_axis_name="s"),
    compiler_params=pltpu.CompilerParams(use_tc_tiling_on_sc=False, needs_layout_passes=False),
    scratch_types=[
        pltpu.VMEM((CTX, B), jnp.int32),
        pltpu.VMEM((NCHUNK, IPC), jnp.int32),
        pltpu.VMEM((IPC, D), jnp.float32),
        pltpu.VMEM((IPC, D), jnp.float32),
        pltpu.VMEM((RPW, D), jnp.float32),
        pltpu.SemaphoreType.DMA,
        pltpu.SemaphoreType.DMA,
    ],
)(_sc_pool_body)


BN = 1024
NB = (V + BN - 1) // BN


def _mm_body(wt_ref, p_ref, b_ref, o_ref):
    o_ref[...] = lax.dot_general(
        wt_ref[...], p_ref[...],
        (((0,), (1,)), ((), ())),
        preferred_element_type=jnp.float32,
    ) + jnp.transpose(b_ref[...])


def _tc_linear(W_t, pooled, b2):
    return pl.pallas_call(
        _mm_body,
        grid=(NB,),
        in_specs=[
            pl.BlockSpec((D, BN), lambda i: (0, i)),
            pl.BlockSpec((B, D), lambda i: (0, 0)),
            pl.BlockSpec((1, BN), lambda i: (0, i)),
        ],
        out_specs=pl.BlockSpec((BN, B), lambda i: (i, 0)),
        out_shape=jax.ShapeDtypeStruct((V, B), jnp.float32),
    )(W_t, pooled, b2)


def kernel(context, emb_table, W, b):
    ctx_t = context.astype(jnp.int32).T
    pooled = _sc_pool(ctx_t, emb_table)
    out_t = _tc_linear(W.T, pooled, b.reshape(1, V))
    return out_t.T


# speedup vs baseline: 1.0197x; 1.0080x over previous
import functools

import numpy as _np

import jax
import jax.numpy as jnp
from jax import lax
from jax.experimental import pallas as pl
from jax.experimental.pallas import tpu as pltpu
from jax.experimental.pallas import tpu_sc as plsc

B = 4096
CTX = 20
D = 64
V = 100000

NC = 2
NS = 16
NW = NC * NS
RPW = B // NW
RPC = 4
IPC = RPC * CTX
NCHUNK = RPW // RPC
LG = D // 16


def _sc_pool_body(ctx_hbm, table_hbm, out_hbm, idx_v, rows0, rows1,
                  pooled_v, sem0, sem1):
    wid = lax.axis_index("s") * NC + lax.axis_index("c")
    pltpu.sync_copy(ctx_hbm.at[wid], idx_v)

    def accum(rows_v, c):
        for r in range(RPC):
            for g in range(LG):
                sl = pl.ds(g * 16, 16)
                acc = rows_v[r * CTX, sl]
                for j in range(1, CTX):
                    acc = acc + rows_v[r * CTX + j, sl]
                pooled_v[c * RPC + r, sl] = acc * (1.0 / CTX)

    def start(c, rows_v, sem):
        return pltpu.async_copy(table_hbm.at[idx_v.at[c]], rows_v, sem)

    start(0, rows0, sem0)
    start(1, rows1, sem1)

    def pair(i, carry):
        pltpu.make_async_copy(
            table_hbm.at[idx_v.at[0]], rows0, sem0).wait()
        accum(rows0, 2 * i)
        start(2 * i + 2, rows0, sem0)
        pltpu.make_async_copy(
            table_hbm.at[idx_v.at[0]], rows1, sem1).wait()
        accum(rows1, 2 * i + 1)
        start(2 * i + 3, rows1, sem1)
        return carry

    lax.fori_loop(0, NCHUNK // 2 - 1, pair, 0)
    pltpu.make_async_copy(table_hbm.at[idx_v.at[0]], rows0, sem0).wait()
    accum(rows0, NCHUNK - 2)
    pltpu.make_async_copy(table_hbm.at[idx_v.at[0]], rows1, sem1).wait()
    accum(rows1, NCHUNK - 1)
    pltpu.sync_copy(pooled_v, out_hbm.at[pl.ds(wid * RPW, RPW)])


_sc_pool = functools.partial(
    pl.kernel,
    out_type=jax.ShapeDtypeStruct((B, D), jnp.float32),
    mesh=plsc.VectorSubcoreMesh(core_axis_name="c", subcore_axis_name="s"),
    compiler_params=pltpu.CompilerParams(use_tc_tiling_on_sc=False),
    scratch_types=[
        pltpu.VMEM((NCHUNK, IPC), jnp.int32),
        pltpu.VMEM((IPC, D), jnp.float32),
        pltpu.VMEM((IPC, D), jnp.float32),
        pltpu.VMEM((RPW, D), jnp.float32),
        pltpu.SemaphoreType.DMA,
        pltpu.SemaphoreType.DMA,
    ],
)(_sc_pool_body)


BN = 512
NB = (V + BN - 1) // BN


def _mm_body(wt_ref, p_ref, b_ref, o_ref):
    o_ref[...] = lax.dot_general(
        wt_ref[...], p_ref[...],
        (((0,), (1,)), ((), ())),
        preferred_element_type=jnp.float32,
    ) + jnp.transpose(b_ref[...])


def _tc_linear(W_t, pooled, b2):
    return pl.pallas_call(
        _mm_body,
        grid=(NB,),
        in_specs=[
            pl.BlockSpec((D, BN), lambda i: (0, i)),
            pl.BlockSpec((B, D), lambda i: (0, 0)),
            pl.BlockSpec((1, BN), lambda i: (0, i)),
        ],
        out_specs=pl.BlockSpec((BN, B), lambda i: (i, 0)),
        out_shape=jax.ShapeDtypeStruct((V, B), jnp.float32),
    )(W_t, pooled, b2)


def kernel(context, emb_table, W, b):
    ctx3 = context.astype(jnp.int32).reshape(NW, NCHUNK, IPC)
    pooled = _sc_pool(ctx3, emb_table)
    out_t = _tc_linear(W.T, pooled, b.reshape(1, V))
    return out_t.T
